# baseline (speedup 1.0000x reference)
import jax
import jax.numpy as jnp
from jax.experimental import pallas as pl
from jax.experimental.pallas import tpu as pltpu

_BLK = 512
_EPS = 1e-05


def _corr_body(q_ref, k_ref, tn_ref, corr_ref):
    q = q_ref[0]
    k = k_ref[0]
    tn = tn_ref[0]
    yy = jnp.sum(k * k, axis=0, keepdims=True)
    xx = jnp.sum(q * q, axis=0, keepdims=True)
    k_aug = jnp.concatenate([2.0 * k, yy], axis=0)
    q_aug = jnp.concatenate(
        [q, jnp.full((1, q.shape[1]), -1.0, dtype=q.dtype)], axis=0)
    lt = jax.lax.dot_general(
        k_aug, q_aug, (((0,), (0,)), ((), ())),
        preferred_element_type=jnp.float32)
    lt = lt - xx
    m = jnp.max(lt, axis=0, keepdims=True)
    p = jnp.exp(lt - m)
    s = jnp.sum(p, axis=0, keepdims=True)
    scores = p / s
    corr_ref[0] = jax.lax.dot_general(
        tn, scores, (((0,), (0,)), ((), ())),
        preferred_element_type=jnp.float32)


def _g(mat, i, j):
    return mat[i:i + 1, j:j + 1]


def _col(mat, j):
    return mat[:, j:j + 1]


def _e_row(j, dtype):
    lane = jax.lax.broadcasted_iota(jnp.int32, (1, 3), 1)
    return jnp.where(lane == j, 1.0, 0.0).astype(dtype)


def _jacobi_rot(A, V, p, q):
    ep = _e_row(p, A.dtype)
    eq = _e_row(q, A.dtype)
    ap = _col(A, p)
    aq = _col(A, q)
    vp = _col(V, p)
    vq = _col(V, q)
    alpha = jnp.sum(ap * ap, axis=0, keepdims=True)
    beta = jnp.sum(aq * aq, axis=0, keepdims=True)
    gamma = jnp.sum(ap * aq, axis=0, keepdims=True)
    absg = jnp.abs(gamma)
    safe_g = jnp.where(absg > 0, gamma, 1.0)
    tau = (beta - alpha) / (2.0 * safe_g)
    sign = jnp.where(tau >= 0, 1.0, -1.0)
    t = sign / (jnp.abs(tau) + jnp.sqrt(1.0 + tau * tau))
    t = jnp.where(absg > 0, t, 0.0)
    c = jax.lax.rsqrt(1.0 + t * t)
    s = t * c
    ap2 = c * ap - s * aq
    aq2 = s * ap + c * aq
    vp2 = c * vp - s * vq
    vq2 = s * vp + c * vq
    A2 = A + (ap2 - ap) * ep + (aq2 - aq) * eq
    V2 = V + (vp2 - vp) * ep + (vq2 - vq) * eq
    return A2, V2


def _pose_body(an_ref, b_ref, rot_ref, tr_ref):
    n = an_ref.shape[-2]
    w = (1.0 / n) / (1.0 + _EPS)
    an = an_ref[0]
    b = b_ref[0]
    ca_row = jnp.sum(an, axis=0, keepdims=True) * w
    cb_col = jnp.sum(b, axis=1, keepdims=True) * w
    ones_row = jnp.full((1, n), 1.0, dtype=b.dtype)
    cb_row = jax.lax.dot_general(
        ones_row, b, (((1,), (1,)), ((), ())),
        preferred_element_type=jnp.float32) * w
    ac = an - ca_row
    bc = b - cb_col
    cov = jax.lax.dot_general(
        ac, bc, (((0,), (1,)), ((), ())),
        preferred_element_type=jnp.float32) * w

    A = cov
    row_i = jax.lax.broadcasted_iota(jnp.int32, (3, 3), 0)
    col_i = jax.lax.broadcasted_iota(jnp.int32, (3, 3), 1)
    V = jnp.where(row_i == col_i, 1.0, 0.0).astype(cov.dtype)
    for _ in range(6):
        for (p, q) in ((0, 1), (0, 2), (1, 2)):
            A, V = _jacobi_rot(A, V, p, q)
    s2 = jnp.sum(A * A, axis=0, keepdims=True)
    s = jnp.sqrt(s2)
    U = A / s
    rot_pos = jax.lax.dot_general(V, U, (((1,), (1,)), ((), ())))
    s0, s1, s2v = _g(s2, 0, 0), _g(s2, 0, 1), _g(s2, 0, 2)
    m0 = jnp.where(jnp.logical_and(s0 <= s1, s0 <= s2v), 1.0, 0.0)
    m1 = jnp.where(jnp.logical_and(s1 < s0, s1 <= s2v), 1.0, 0.0)
    m2 = jnp.where(jnp.logical_and(s2v < s0, s2v < s1), 1.0, 0.0)
    sel = (m0 * _e_row(0, cov.dtype)
           + m1 * _e_row(1, cov.dtype)
           + m2 * _e_row(2, cov.dtype))
    v3 = jax.lax.dot_general(V, sel, (((1,), (1,)), ((), ())))
    u3t = jax.lax.dot_general(sel, U, (((1,), (1,)), ((), ())))
    rot_neg = rot_pos - 2.0 * (v3 * u3t)
    det = (_g(cov, 0, 0) * (_g(cov, 1, 1) * _g(cov, 2, 2) - _g(cov, 1, 2) * _g(cov, 2, 1))
           - _g(cov, 0, 1) * (_g(cov, 1, 0) * _g(cov, 2, 2) - _g(cov, 1, 2) * _g(cov, 2, 0))
           + _g(cov, 0, 2) * (_g(cov, 1, 0) * _g(cov, 2, 1) - _g(cov, 1, 1) * _g(cov, 2, 0)))
    pos_w = jnp.where(det > 0, 1.0, 0.0)
    rot = rot_neg + pos_w * (rot_pos - rot_neg)
    trans = cb_row - jax.lax.dot_general(
        ca_row, rot, (((1,), (1,)), ((), ())))
    rot_ref[0] = rot
    tr_ref[0] = trans


def kernel(src, tgt, src_embedding, tgt_embedding, positive_T):
    batch, posi_num, num_points, cdim = tgt.shape
    bp = batch * posi_num
    cemb = tgt_embedding.shape[2]

    src_ = jnp.swapaxes(src, -2, -1).reshape(bp, cdim, num_points)
    src_nat = src.reshape(bp, num_points, cdim)
    tgt_nat = tgt.reshape(bp, num_points, cdim)
    src_emb = jnp.squeeze(src_embedding, -1).reshape(batch, cemb, num_points)
    tgt_emb = jnp.squeeze(tgt_embedding, -1).reshape(bp, cemb, num_points)

    nblk = num_points // _BLK
    corr = pl.pallas_call(
        _corr_body,
        grid=(bp, nblk),
        in_specs=[
            pl.BlockSpec((1, cemb, _BLK),
                         lambda b, j, _p=posi_num: (b // _p, 0, j)),
            pl.BlockSpec((1, cemb, num_points), lambda b, j: (b, 0, 0)),
            pl.BlockSpec((1, num_points, cdim), lambda b, j: (b, 0, 0)),
        ],
        out_specs=pl.BlockSpec((1, cdim, _BLK), lambda b, j: (b, 0, j)),
        out_shape=jax.ShapeDtypeStruct((bp, cdim, num_points), jnp.float32),
        compiler_params=pltpu.CompilerParams(
            dimension_semantics=("parallel", "parallel")),
    )(src_emb, tgt_emb, tgt_nat)

    rot_mat, trans = pl.pallas_call(
        _pose_body,
        grid=(bp,),
        in_specs=[
            pl.BlockSpec((1, num_points, cdim), lambda b: (b, 0, 0)),
            pl.BlockSpec((1, cdim, num_points), lambda b: (b, 0, 0)),
        ],
        out_specs=[
            pl.BlockSpec((1, cdim, cdim), lambda b: (b, 0, 0)),
            pl.BlockSpec((1, 1, cdim), lambda b: (b, 0, 0)),
        ],
        out_shape=[
            jax.ShapeDtypeStruct((bp, cdim, cdim), jnp.float32),
            jax.ShapeDtypeStruct((bp, 1, cdim), jnp.float32),
        ],
        compiler_params=pltpu.CompilerParams(
            dimension_semantics=("arbitrary",)),
    )(src_nat, corr)

    translation = trans.reshape(bp, cdim)
    return (rot_mat, translation, src_, corr)


# speedup vs baseline: 1.1191x; 1.1191x over previous
import jax
import jax.numpy as jnp
from jax.experimental import pallas as pl
from jax.experimental.pallas import tpu as pltpu

_BLK = 512
_EPS = 1e-05


def _corr_body(qt_ref, ke_ref, v_ref, corr_ref):
    qt = qt_ref[0]
    k = ke_ref[0]
    v = v_ref[0]
    inner = -2.0 * jax.lax.dot_general(
        qt, k, (((1,), (0,)), ((), ())),
        preferred_element_type=jnp.float32)
    xx = jnp.sum(qt * qt, axis=1, keepdims=True)
    yy = jnp.sum(k * k, axis=0, keepdims=True)
    logits = -xx - inner - yy
    m = jnp.max(logits, axis=1, keepdims=True)
    p = jnp.exp(logits - m)
    s = jnp.sum(p, axis=1, keepdims=True)
    scores = p / s
    corr_ref[0] = jax.lax.dot_general(
        v, scores, (((1,), (1,)), ((), ())),
        preferred_element_type=jnp.float32)


def _g(mat, i, j):
    return mat[i:i + 1, j:j + 1]


def _col(mat, j):
    return mat[:, j:j + 1]


def _e_row(j, dtype):
    lane = jax.lax.broadcasted_iota(jnp.int32, (1, 3), 1)
    return jnp.where(lane == j, 1.0, 0.0).astype(dtype)


def _jacobi_rot(A, V, p, q):
    ep = _e_row(p, A.dtype)
    eq = _e_row(q, A.dtype)
    ap = _col(A, p)
    aq = _col(A, q)
    vp = _col(V, p)
    vq = _col(V, q)
    alpha = jnp.sum(ap * ap, axis=0, keepdims=True)
    beta = jnp.sum(aq * aq, axis=0, keepdims=True)
    gamma = jnp.sum(ap * aq, axis=0, keepdims=True)
    absg = jnp.abs(gamma)
    safe_g = jnp.where(absg > 0, gamma, 1.0)
    tau = (beta - alpha) / (2.0 * safe_g)
    sign = jnp.where(tau >= 0, 1.0, -1.0)
    t = sign / (jnp.abs(tau) + jnp.sqrt(1.0 + tau * tau))
    t = jnp.where(absg > 0, t, 0.0)
    c = jax.lax.rsqrt(1.0 + t * t)
    s = t * c
    ap2 = c * ap - s * aq
    aq2 = s * ap + c * aq
    vp2 = c * vp - s * vq
    vq2 = s * vp + c * vq
    A2 = A + (ap2 - ap) * ep + (aq2 - aq) * eq
    V2 = V + (vp2 - vp) * ep + (vq2 - vq) * eq
    return A2, V2


def _pose_body(a_ref, b_ref, rot_ref, tr_ref):
    n = a_ref.shape[-1]
    w = (1.0 / n) / (1.0 + _EPS)
    a = a_ref[0]
    b = b_ref[0]
    ca = jnp.sum(a, axis=1, keepdims=True) * w
    cb = jnp.sum(b, axis=1, keepdims=True) * w
    ac = a - ca
    bc = b - cb
    cov = jax.lax.dot_general(
        ac, bc, (((1,), (1,)), ((), ())),
        preferred_element_type=jnp.float32) * w

    A = cov
    row_i = jax.lax.broadcasted_iota(jnp.int32, (3, 3), 0)
    col_i = jax.lax.broadcasted_iota(jnp.int32, (3, 3), 1)
    V = jnp.where(row_i == col_i, 1.0, 0.0).astype(cov.dtype)
    for _ in range(4):
        for (p, q) in ((0, 1), (0, 2), (1, 2)):
            A, V = _jacobi_rot(A, V, p, q)
    s2 = jnp.sum(A * A, axis=0, keepdims=True)
    s = jnp.sqrt(s2)
    U = A / s
    rot_pos = jax.lax.dot_general(V, U, (((1,), (1,)), ((), ())))
    s0, s1, s2v = _g(s2, 0, 0), _g(s2, 0, 1), _g(s2, 0, 2)
    m0 = jnp.where(jnp.logical_and(s0 <= s1, s0 <= s2v), 1.0, 0.0)
    m1 = jnp.where(jnp.logical_and(s1 < s0, s1 <= s2v), 1.0, 0.0)
    m2 = jnp.where(jnp.logical_and(s2v < s0, s2v < s1), 1.0, 0.0)
    sel = (m0 * _e_row(0, cov.dtype)
           + m1 * _e_row(1, cov.dtype)
           + m2 * _e_row(2, cov.dtype))
    v3 = jax.lax.dot_general(V, sel, (((1,), (1,)), ((), ())))
    u3t = jax.lax.dot_general(sel, U, (((1,), (1,)), ((), ())))
    rot_neg = rot_pos - 2.0 * (v3 * u3t)
    det = (_g(cov, 0, 0) * (_g(cov, 1, 1) * _g(cov, 2, 2) - _g(cov, 1, 2) * _g(cov, 2, 1))
           - _g(cov, 0, 1) * (_g(cov, 1, 0) * _g(cov, 2, 2) - _g(cov, 1, 2) * _g(cov, 2, 0))
           + _g(cov, 0, 2) * (_g(cov, 1, 0) * _g(cov, 2, 1) - _g(cov, 1, 1) * _g(cov, 2, 0)))
    pos_w = jnp.where(det > 0, 1.0, 0.0)
    rot = rot_neg + pos_w * (rot_pos - rot_neg)
    trans = cb - jax.lax.dot_general(rot, ca, (((1,), (0,)), ((), ())))
    rot_ref[0] = rot
    tr_ref[0] = trans


def kernel(src, tgt, src_embedding, tgt_embedding, positive_T):
    batch, posi_num, num_points, cdim = tgt.shape
    bp = batch * posi_num
    cemb = tgt_embedding.shape[2]

    src_ = jnp.swapaxes(src, -2, -1).reshape(bp, cdim, num_points)
    tgt_ = jnp.swapaxes(tgt, -2, -1).reshape(bp, cdim, num_points)
    src_emb_t = jnp.broadcast_to(
        jnp.swapaxes(jnp.squeeze(src_embedding, -1), -2, -1),
        (batch, posi_num, num_points, cemb)).reshape(bp, num_points, cemb)
    tgt_emb = jnp.squeeze(tgt_embedding, -1).reshape(bp, cemb, num_points)

    nblk = num_points // _BLK
    corr = pl.pallas_call(
        _corr_body,
        grid=(bp, nblk),
        in_specs=[
            pl.BlockSpec((1, _BLK, cemb), lambda b, j: (b, j, 0)),
            pl.BlockSpec((1, cemb, num_points), lambda b, j: (b, 0, 0)),
            pl.BlockSpec((1, cdim, num_points), lambda b, j: (b, 0, 0)),
        ],
        out_specs=pl.BlockSpec((1, cdim, _BLK), lambda b, j: (b, 0, j)),
        out_shape=jax.ShapeDtypeStruct((bp, cdim, num_points), jnp.float32),
        compiler_params=pltpu.CompilerParams(
            dimension_semantics=("parallel", "parallel")),
    )(src_emb_t, tgt_emb, tgt_)

    rot_mat, trans = pl.pallas_call(
        _pose_body,
        grid=(bp,),
        in_specs=[
            pl.BlockSpec((1, cdim, num_points), lambda b: (b, 0, 0)),
            pl.BlockSpec((1, cdim, num_points), lambda b: (b, 0, 0)),
        ],
        out_specs=[
            pl.BlockSpec((1, cdim, cdim), lambda b: (b, 0, 0)),
            pl.BlockSpec((1, cdim, 1), lambda b: (b, 0, 0)),
        ],
        out_shape=[
            jax.ShapeDtypeStruct((bp, cdim, cdim), jnp.float32),
            jax.ShapeDtypeStruct((bp, cdim, 1), jnp.float32),
        ],
        compiler_params=pltpu.CompilerParams(
            dimension_semantics=("arbitrary",)),
    )(src_, corr)

    translation = trans.reshape(bp, cdim)
    return (rot_mat, translation, src_, corr)


# speedup vs baseline: 1.4248x; 1.2731x over previous
import jax
import jax.numpy as jnp
from jax.experimental import pallas as pl
from jax.experimental.pallas import tpu as pltpu

_BLK = 512
_EPS = 1e-05


def _corr_body(qt_ref, ke_ref, v_ref, corr_ref):
    qt = qt_ref[0]
    k = ke_ref[0]
    v = v_ref[0]
    inner = -2.0 * jax.lax.dot_general(
        qt, k, (((1,), (0,)), ((), ())),
        preferred_element_type=jnp.float32)
    xx = jnp.sum(qt * qt, axis=1, keepdims=True)
    yy = jnp.sum(k * k, axis=0, keepdims=True)
    logits = -xx - inner - yy
    m = jnp.max(logits, axis=1, keepdims=True)
    p = jnp.exp(logits - m)
    s = jnp.sum(p, axis=1, keepdims=True)
    scores = p / s
    corr_ref[0] = jax.lax.dot_general(
        v, scores, (((1,), (1,)), ((), ())),
        preferred_element_type=jnp.float32)


def _g3(mat, i, j):
    return mat[:, i:i + 1, j:j + 1]


def _e_lane(j, dtype):
    lane = jax.lax.broadcasted_iota(jnp.int32, (1, 1, 3), 2)
    return jnp.where(lane == j, 1.0, 0.0).astype(dtype)


def _e_sub(j, dtype):
    sub = jax.lax.broadcasted_iota(jnp.int32, (1, 3, 1), 1)
    return jnp.where(sub == j, 1.0, 0.0).astype(dtype)


def _jacobi_rot(At, V, p, q):
    atp = At[:, p:p + 1, :]
    atq = At[:, q:q + 1, :]
    vp = V[:, :, p:p + 1]
    vq = V[:, :, q:q + 1]
    alpha = jnp.sum(atp * atp, axis=2, keepdims=True)
    beta = jnp.sum(atq * atq, axis=2, keepdims=True)
    gamma = jnp.sum(atp * atq, axis=2, keepdims=True)
    absg = jnp.abs(gamma)
    safe_g = jnp.where(absg > 0, gamma, 1.0)
    tau = (beta - alpha) / (2.0 * safe_g)
    sign = jnp.where(tau >= 0, 1.0, -1.0)
    t = sign / (jnp.abs(tau) + jnp.sqrt(1.0 + tau * tau))
    t = jnp.where(absg > 0, t, 0.0)
    c = jax.lax.rsqrt(1.0 + t * t)
    s = t * c
    atp2 = c * atp - s * atq
    atq2 = s * atp + c * atq
    vp2 = c * vp - s * vq
    vq2 = s * vp + c * vq
    At2 = At + (atp2 - atp) * _e_sub(p, At.dtype) + (atq2 - atq) * _e_sub(q, At.dtype)
    V2 = V + (vp2 - vp) * _e_lane(p, V.dtype) + (vq2 - vq) * _e_lane(q, V.dtype)
    return At2, V2


def _pose_body(a_ref, b_ref, rot_ref, tr_ref):
    bp = a_ref.shape[0]
    n = a_ref.shape[-1]
    dt = a_ref.dtype
    w = (1.0 / n) / (1.0 + _EPS)
    a = a_ref[...]
    b = b_ref[...]
    ca = jnp.sum(a, axis=2, keepdims=True) * w
    cb = jnp.sum(b, axis=2, keepdims=True) * w
    ac = a - ca
    bc = b - cb
    covT = jnp.zeros((bp, 3, 3), dtype=dt)
    for i in range(bp):
        ct = jax.lax.dot_general(
            bc[i], ac[i], (((1,), (1,)), ((), ())),
            preferred_element_type=jnp.float32) * w
        bmask = jnp.where(
            jax.lax.broadcasted_iota(jnp.int32, (bp, 1, 1), 0) == i, 1.0, 0.0
        ).astype(dt)
        covT = covT + bmask * ct[None]

    At = covT
    row_i = jax.lax.broadcasted_iota(jnp.int32, (1, 3, 3), 1)
    col_i = jax.lax.broadcasted_iota(jnp.int32, (1, 3, 3), 2)
    eye = jnp.where(row_i == col_i, 1.0, 0.0).astype(dt)
    V = jnp.zeros((bp, 3, 3), dtype=dt) + eye
    for _ in range(4):
        for (p, q) in ((0, 1), (0, 2), (1, 2)):
            At, V = _jacobi_rot(At, V, p, q)
    s2 = jnp.sum(At * At, axis=2, keepdims=True)
    Ut = At / jnp.sqrt(s2)
    rot_pos = (V[:, :, 0:1] * Ut[:, 0:1, :]
               + V[:, :, 1:2] * Ut[:, 1:2, :]
               + V[:, :, 2:3] * Ut[:, 2:3, :])
    s0 = s2[:, 0:1, :]
    s1 = s2[:, 1:2, :]
    s2v = s2[:, 2:3, :]
    m0 = jnp.where(jnp.logical_and(s0 <= s1, s0 <= s2v), 1.0, 0.0)
    m1 = jnp.where(jnp.logical_and(s1 < s0, s1 <= s2v), 1.0, 0.0)
    m2 = jnp.where(jnp.logical_and(s2v < s0, s2v < s1), 1.0, 0.0)
    v3 = m0 * V[:, :, 0:1] + m1 * V[:, :, 1:2] + m2 * V[:, :, 2:3]
    u3t = m0 * Ut[:, 0:1, :] + m1 * Ut[:, 1:2, :] + m2 * Ut[:, 2:3, :]
    rot_neg = rot_pos - 2.0 * (v3 * u3t)
    det = (_g3(covT, 0, 0) * (_g3(covT, 1, 1) * _g3(covT, 2, 2) - _g3(covT, 2, 1) * _g3(covT, 1, 2))
           - _g3(covT, 1, 0) * (_g3(covT, 0, 1) * _g3(covT, 2, 2) - _g3(covT, 2, 1) * _g3(covT, 0, 2))
           + _g3(covT, 2, 0) * (_g3(covT, 0, 1) * _g3(covT, 1, 2) - _g3(covT, 1, 1) * _g3(covT, 0, 2)))
    pos_w = jnp.where(det > 0, 1.0, 0.0)
    rot = rot_neg + pos_w * (rot_pos - rot_neg)
    ca_row = ((jnp.sum(ca * _e_sub(0, dt), axis=1, keepdims=True)) * _e_lane(0, dt)
              + (jnp.sum(ca * _e_sub(1, dt), axis=1, keepdims=True)) * _e_lane(1, dt)
              + (jnp.sum(ca * _e_sub(2, dt), axis=1, keepdims=True)) * _e_lane(2, dt))
    rot_ca = jnp.sum(rot * ca_row, axis=2, keepdims=True)
    trans = cb - rot_ca
    rot_ref[...] = rot
    tr_ref[...] = trans


def kernel(src, tgt, src_embedding, tgt_embedding, positive_T):
    batch, posi_num, num_points, cdim = tgt.shape
    bp = batch * posi_num
    cemb = tgt_embedding.shape[2]

    src_ = jnp.swapaxes(src, -2, -1).reshape(bp, cdim, num_points)
    tgt_ = jnp.swapaxes(tgt, -2, -1).reshape(bp, cdim, num_points)
    src_emb_t = jnp.broadcast_to(
        jnp.swapaxes(jnp.squeeze(src_embedding, -1), -2, -1),
        (batch, posi_num, num_points, cemb)).reshape(bp, num_points, cemb)
    tgt_emb = jnp.squeeze(tgt_embedding, -1).reshape(bp, cemb, num_points)

    nblk = num_points // _BLK
    corr = pl.pallas_call(
        _corr_body,
        grid=(bp, nblk),
        in_specs=[
            pl.BlockSpec((1, _BLK, cemb), lambda b, j: (b, j, 0)),
            pl.BlockSpec((1, cemb, num_points), lambda b, j: (b, 0, 0)),
            pl.BlockSpec((1, cdim, num_points), lambda b, j: (b, 0, 0)),
        ],
        out_specs=pl.BlockSpec((1, cdim, _BLK), lambda b, j: (b, 0, j)),
        out_shape=jax.ShapeDtypeStruct((bp, cdim, num_points), jnp.float32),
        compiler_params=pltpu.CompilerParams(
            dimension_semantics=("parallel", "parallel")),
    )(src_emb_t, tgt_emb, tgt_)

    rot_mat, trans = pl.pallas_call(
        _pose_body,
        grid=(1,),
        in_specs=[
            pl.BlockSpec((bp, cdim, num_points), lambda i: (0, 0, 0)),
            pl.BlockSpec((bp, cdim, num_points), lambda i: (0, 0, 0)),
        ],
        out_specs=[
            pl.BlockSpec((bp, cdim, cdim), lambda i: (0, 0, 0)),
            pl.BlockSpec((bp, cdim, 1), lambda i: (0, 0, 0)),
        ],
        out_shape=[
            jax.ShapeDtypeStruct((bp, cdim, cdim), jnp.float32),
            jax.ShapeDtypeStruct((bp, cdim, 1), jnp.float32),
        ],
        compiler_params=pltpu.CompilerParams(
            dimension_semantics=("arbitrary",)),
    )(src_, corr)

    translation = trans.reshape(bp, cdim)
    return (rot_mat, translation, src_, corr)


# speedup vs baseline: 1.4636x; 1.0273x over previous
import jax
import jax.numpy as jnp
from jax.experimental import pallas as pl
from jax.experimental.pallas import tpu as pltpu

_BLK = 512
_EPS = 1e-05


def _corr_body(qt_ref, ke_ref, v_ref, corr_ref):
    qt = qt_ref[0]
    k = ke_ref[0]
    v = v_ref[0]
    inner = -2.0 * jax.lax.dot_general(
        qt, k, (((1,), (0,)), ((), ())),
        preferred_element_type=jnp.float32)
    xx = jnp.sum(qt * qt, axis=1, keepdims=True)
    yy = jnp.sum(k * k, axis=0, keepdims=True)
    logits = -xx - inner - yy
    m = jnp.max(logits, axis=1, keepdims=True)
    p = jnp.exp(logits - m)
    s = jnp.sum(p, axis=1, keepdims=True)
    scores = p / s
    corr_ref[0] = jax.lax.dot_general(
        v, scores, (((1,), (1,)), ((), ())),
        preferred_element_type=jnp.float32)


def _g3(mat, i, j):
    return mat[:, i:i + 1, j:j + 1]


def _e_lane(j, dtype):
    lane = jax.lax.broadcasted_iota(jnp.int32, (1, 1, 3), 2)
    return jnp.where(lane == j, 1.0, 0.0).astype(dtype)


def _e_sub(j, dtype):
    sub = jax.lax.broadcasted_iota(jnp.int32, (1, 3, 1), 1)
    return jnp.where(sub == j, 1.0, 0.0).astype(dtype)


def _jacobi_rot(At, V, p, q):
    atp = At[:, p:p + 1, :]
    atq = At[:, q:q + 1, :]
    vp = V[:, :, p:p + 1]
    vq = V[:, :, q:q + 1]
    alpha = jnp.sum(atp * atp, axis=2, keepdims=True)
    beta = jnp.sum(atq * atq, axis=2, keepdims=True)
    gamma = jnp.sum(atp * atq, axis=2, keepdims=True)
    absg = jnp.abs(gamma)
    safe_g = jnp.where(absg > 0, gamma, 1.0)
    tau = (beta - alpha) / (2.0 * safe_g)
    sign = jnp.where(tau >= 0, 1.0, -1.0)
    t = sign / (jnp.abs(tau) + jnp.sqrt(1.0 + tau * tau))
    t = jnp.where(absg > 0, t, 0.0)
    c = jax.lax.rsqrt(1.0 + t * t)
    s = t * c
    atp2 = c * atp - s * atq
    atq2 = s * atp + c * atq
    vp2 = c * vp - s * vq
    vq2 = s * vp + c * vq
    At2 = At + (atp2 - atp) * _e_sub(p, At.dtype) + (atq2 - atq) * _e_sub(q, At.dtype)
    V2 = V + (vp2 - vp) * _e_lane(p, V.dtype) + (vq2 - vq) * _e_lane(q, V.dtype)
    return At2, V2


def _pose_body(a_ref, b_ref, rot_ref, tr_ref):
    bp = a_ref.shape[0]
    n = a_ref.shape[-1]
    dt = a_ref.dtype
    w = (1.0 / n) / (1.0 + _EPS)
    a = a_ref[...]
    b = b_ref[...]
    ca = jnp.sum(a, axis=2, keepdims=True) * w
    cb = jnp.sum(b, axis=2, keepdims=True) * w
    ac = a - ca
    bc = b - cb
    covT = jnp.zeros((bp, 3, 3), dtype=dt)
    for i in range(bp):
        ct = jax.lax.dot_general(
            bc[i], ac[i], (((1,), (1,)), ((), ())),
            preferred_element_type=jnp.float32) * w
        bmask = jnp.where(
            jax.lax.broadcasted_iota(jnp.int32, (bp, 1, 1), 0) == i, 1.0, 0.0
        ).astype(dt)
        covT = covT + bmask * ct[None]

    At = covT
    row_i = jax.lax.broadcasted_iota(jnp.int32, (1, 3, 3), 1)
    col_i = jax.lax.broadcasted_iota(jnp.int32, (1, 3, 3), 2)
    eye = jnp.where(row_i == col_i, 1.0, 0.0).astype(dt)
    V = jnp.zeros((bp, 3, 3), dtype=dt) + eye
    for _ in range(4):
        for (p, q) in ((0, 1), (0, 2), (1, 2)):
            At, V = _jacobi_rot(At, V, p, q)
    s2 = jnp.sum(At * At, axis=2, keepdims=True)
    Ut = At / jnp.sqrt(s2)
    rot_pos = (V[:, :, 0:1] * Ut[:, 0:1, :]
               + V[:, :, 1:2] * Ut[:, 1:2, :]
               + V[:, :, 2:3] * Ut[:, 2:3, :])
    s0 = s2[:, 0:1, :]
    s1 = s2[:, 1:2, :]
    s2v = s2[:, 2:3, :]
    m0 = jnp.where(jnp.logical_and(s0 <= s1, s0 <= s2v), 1.0, 0.0)
    m1 = jnp.where(jnp.logical_and(s1 < s0, s1 <= s2v), 1.0, 0.0)
    m2 = jnp.where(jnp.logical_and(s2v < s0, s2v < s1), 1.0, 0.0)
    v3 = m0 * V[:, :, 0:1] + m1 * V[:, :, 1:2] + m2 * V[:, :, 2:3]
    u3t = m0 * Ut[:, 0:1, :] + m1 * Ut[:, 1:2, :] + m2 * Ut[:, 2:3, :]
    rot_neg = rot_pos - 2.0 * (v3 * u3t)
    det = (_g3(covT, 0, 0) * (_g3(covT, 1, 1) * _g3(covT, 2, 2) - _g3(covT, 2, 1) * _g3(covT, 1, 2))
           - _g3(covT, 1, 0) * (_g3(covT, 0, 1) * _g3(covT, 2, 2) - _g3(covT, 2, 1) * _g3(covT, 0, 2))
           + _g3(covT, 2, 0) * (_g3(covT, 0, 1) * _g3(covT, 1, 2) - _g3(covT, 1, 1) * _g3(covT, 0, 2)))
    pos_w = jnp.where(det > 0, 1.0, 0.0)
    rot = rot_neg + pos_w * (rot_pos - rot_neg)
    ca_row = ((jnp.sum(ca * _e_sub(0, dt), axis=1, keepdims=True)) * _e_lane(0, dt)
              + (jnp.sum(ca * _e_sub(1, dt), axis=1, keepdims=True)) * _e_lane(1, dt)
              + (jnp.sum(ca * _e_sub(2, dt), axis=1, keepdims=True)) * _e_lane(2, dt))
    rot_ca = jnp.sum(rot * ca_row, axis=2, keepdims=True)
    trans = cb - rot_ca
    rot_ref[...] = rot
    tr_ref[...] = trans


def kernel(src, tgt, src_embedding, tgt_embedding, positive_T):
    batch, posi_num, num_points, cdim = tgt.shape
    bp = batch * posi_num
    cemb = tgt_embedding.shape[2]

    src_ = jnp.swapaxes(src, -2, -1).reshape(bp, cdim, num_points)
    tgt_ = jnp.swapaxes(tgt, -2, -1).reshape(bp, cdim, num_points)
    src_emb_t = jnp.swapaxes(
        jnp.squeeze(src_embedding, -1), -2, -1).reshape(
            batch, num_points, cemb)
    tgt_emb = jnp.squeeze(tgt_embedding, -1).reshape(bp, cemb, num_points)

    nblk = num_points // _BLK
    corr = pl.pallas_call(
        _corr_body,
        grid=(bp, nblk),
        in_specs=[
            pl.BlockSpec((1, _BLK, cemb),
                         lambda b, j, _p=posi_num: (b // _p, j, 0)),
            pl.BlockSpec((1, cemb, num_points), lambda b, j: (b, 0, 0)),
            pl.BlockSpec((1, cdim, num_points), lambda b, j: (b, 0, 0)),
        ],
        out_specs=pl.BlockSpec((1, cdim, _BLK), lambda b, j: (b, 0, j)),
        out_shape=jax.ShapeDtypeStruct((bp, cdim, num_points), jnp.float32),
        compiler_params=pltpu.CompilerParams(
            dimension_semantics=("parallel", "parallel")),
    )(src_emb_t, tgt_emb, tgt_)

    rot_mat, trans = pl.pallas_call(
        _pose_body,
        grid=(1,),
        in_specs=[
            pl.BlockSpec((bp, cdim, num_points), lambda i: (0, 0, 0)),
            pl.BlockSpec((bp, cdim, num_points), lambda i: (0, 0, 0)),
        ],
        out_specs=[
            pl.BlockSpec((bp, cdim, cdim), lambda i: (0, 0, 0)),
            pl.BlockSpec((bp, cdim, 1), lambda i: (0, 0, 0)),
        ],
        out_shape=[
            jax.ShapeDtypeStruct((bp, cdim, cdim), jnp.float32),
            jax.ShapeDtypeStruct((bp, cdim, 1), jnp.float32),
        ],
        compiler_params=pltpu.CompilerParams(
            dimension_semantics=("arbitrary",)),
    )(src_, corr)

    translation = trans.reshape(bp, cdim)
    return (rot_mat, translation, src_, corr)
